# baseline (speedup 1.0000x reference)
import functools

import jax
import jax.numpy as jnp
from jax import lax
from jax.experimental import pallas as pl
from jax.experimental.pallas import tpu as pltpu
from jax.experimental.pallas import tpu_sc as plsc

POOL = 10000
BATCH = 1024
SEL = 5
PLEN = 8
DIM = 768
NROWS = BATCH * SEL
NW = 32
PER_W = NROWS // NW
GROUP = 16
NGROUPS = PER_W // GROUP


@functools.partial(
    pl.kernel,
    mesh=plsc.VectorSubcoreMesh(core_axis_name="c", subcore_axis_name="s"),
    out_type=jax.ShapeDtypeStruct((NROWS, 3 * PLEN, DIM), jnp.float32),
    scratch_types=[
        pltpu.VMEM((PER_W,), jnp.int32),
        pltpu.SemaphoreType.DMA,
    ],
)
def _gather3(idx_hbm, a_hbm, b_hbm, c_hbm, out_hbm, idx_v, sem):
    wid = lax.axis_index("s") * 2 + lax.axis_index("c")
    base = wid * PER_W
    pltpu.sync_copy(idx_hbm.at[pl.ds(base, PER_W)], idx_v)

    def issue_group(g):
        vec = idx_v[pl.ds(g * GROUP, GROUP)]
        for r in range(GROUP):
            v = vec[r]
            i = g * GROUP + r
            for t, tab in enumerate((a_hbm, b_hbm, c_hbm)):
                pltpu.async_copy(
                    tab.at[pl.ds(v, 1)],
                    out_hbm.at[pl.ds(base + i, 1), pl.ds(t * PLEN, PLEN), :],
                    sem,
                )

    def drain_group():
        for _ in range(GROUP * 3):
            pltpu.make_async_copy(
                a_hbm.at[pl.ds(0, 1)],
                out_hbm.at[pl.ds(base, 1), pl.ds(0, PLEN), :],
                sem,
            ).wait()

    issue_group(0)

    def body(g, _):
        issue_group(g)
        drain_group()
        return ()

    lax.fori_loop(1, NGROUPS, body, ())
    drain_group()


def kernel(indices, part_A, part_B, part_C):
    idx = indices.reshape(NROWS).astype(jnp.int32)
    out = _gather3(idx, part_A, part_B, part_C)
    return out.reshape(BATCH, SEL, 3 * PLEN, DIM)


# speedup vs baseline: 41.1396x; 41.1396x over previous
import functools

import jax
import jax.numpy as jnp
from jax import lax
from jax.experimental import pallas as pl
from jax.experimental.pallas import tpu as pltpu
from jax.experimental.pallas import tpu_sc as plsc

POOL = 10000
BATCH = 1024
SEL = 5
PLEN = 8
DIM = 768
NROWS = BATCH * SEL
NW = 32
PER_W = NROWS // NW
CHUNK = 4
NCHUNKS = PER_W // CHUNK
RING = 5
STEPS = NCHUNKS // RING


@functools.partial(
    pl.kernel,
    mesh=plsc.VectorSubcoreMesh(core_axis_name="c", subcore_axis_name="s"),
    out_type=jax.ShapeDtypeStruct((NROWS, 3 * PLEN, DIM), jnp.float32),
    scratch_types=[
        pltpu.VMEM((NCHUNKS, CHUNK), jnp.int32),
        pltpu.VMEM((RING, CHUNK, PLEN, DIM), jnp.float32),
        pltpu.SemaphoreType.DMA,
        pltpu.SemaphoreType.DMA,
    ],
)
def _gather3(idx_hbm, a_hbm, b_hbm, c_hbm, out_hbm, idx_v, bufs, gsem, wsem):
    wid = lax.axis_index("s") * 2 + lax.axis_index("c")
    base = wid * PER_W
    pltpu.sync_copy(idx_hbm.at[wid], idx_v)

    def out_slice(jb, t):
        return out_hbm.at[
            pl.ds(base + jb * CHUNK, CHUNK), pl.ds(t * PLEN, PLEN), :
        ]

    for t, tab in enumerate((a_hbm, b_hbm, c_hbm)):
        def body(step, _, tab=tab, t=t):
            jb0 = step * RING

            gathers = []
            for b in range(RING):
                @pl.when(step > 0)
                def _():
                    pltpu.make_async_copy(bufs.at[b], out_slice(0, t), wsem).wait()

                gathers.append(
                    pltpu.async_copy(tab.at[idx_v.at[jb0 + b]], bufs.at[b], gsem)
                )
            for b in range(RING):
                gathers[b].wait()
                pltpu.async_copy(bufs.at[b], out_slice(jb0 + b, t), wsem)
            return ()

        lax.fori_loop(0, STEPS, body, ())
        for b in range(RING):
            pltpu.make_async_copy(bufs.at[b], out_slice(0, t), wsem).wait()


def kernel(indices, part_A, part_B, part_C):
    idx = indices.reshape(NROWS).astype(jnp.int32).reshape(NW, NCHUNKS, CHUNK)
    out = _gather3(idx, part_A, part_B, part_C)
    return out.reshape(BATCH, SEL, 3 * PLEN, DIM)


# speedup vs baseline: 41.3935x; 1.0062x over previous
import functools

import jax
import jax.numpy as jnp
from jax import lax
from jax.experimental import pallas as pl
from jax.experimental.pallas import tpu as pltpu
from jax.experimental.pallas import tpu_sc as plsc

POOL = 10000
BATCH = 1024
SEL = 5
PLEN = 8
DIM = 768
NROWS = BATCH * SEL
NW = 32
PER_W = NROWS // NW
CHUNK = 4
NCHUNKS = PER_W // CHUNK
RING = 5
STEPS = NCHUNKS // RING


@functools.partial(
    pl.kernel,
    mesh=plsc.VectorSubcoreMesh(core_axis_name="c", subcore_axis_name="s"),
    out_type=jax.ShapeDtypeStruct((NROWS, 3 * PLEN, DIM), jnp.float32),
    scratch_types=[
        pltpu.VMEM((NCHUNKS, CHUNK), jnp.int32),
        pltpu.VMEM((RING, CHUNK, PLEN, DIM), jnp.float32),
        pltpu.SemaphoreType.DMA,
        pltpu.SemaphoreType.DMA,
    ],
)
def _gather3(idx_hbm, a_hbm, b_hbm, c_hbm, out_hbm, idx_v, bufs, gsem, wsem):
    wid = lax.axis_index("s") * 2 + lax.axis_index("c")
    base = wid * PER_W
    pltpu.sync_copy(idx_hbm.at[wid], idx_v)

    def out_slice(jb, t):
        return out_hbm.at[
            pl.ds(base + jb * CHUNK, CHUNK), pl.ds(t * PLEN, PLEN), :
        ]

    for t, tab in enumerate((a_hbm, b_hbm, c_hbm)):
        def body(step, _, tab=tab, t=t):
            jb0 = step * RING

            gathers = []
            for b in range(RING):
                if t == 0:
                    @pl.when(step > 0)
                    def _():
                        pltpu.make_async_copy(
                            bufs.at[b], out_slice(0, t), wsem
                        ).wait()
                else:
                    pltpu.make_async_copy(bufs.at[b], out_slice(0, t), wsem).wait()

                gathers.append(
                    pltpu.async_copy(tab.at[idx_v.at[jb0 + b]], bufs.at[b], gsem)
                )
            for b in range(RING):
                gathers[b].wait()
                pltpu.async_copy(bufs.at[b], out_slice(jb0 + b, t), wsem)
            return ()

        lax.fori_loop(0, STEPS, body, ())

    for b in range(RING):
        pltpu.make_async_copy(bufs.at[b], out_slice(0, 0), wsem).wait()


def kernel(indices, part_A, part_B, part_C):
    idx = indices.reshape(NROWS).astype(jnp.int32).reshape(NW, NCHUNKS, CHUNK)
    out = _gather3(idx, part_A, part_B, part_C)
    return out.reshape(BATCH, SEL, 3 * PLEN, DIM)
